# baseline (speedup 1.0000x reference)
import functools

import jax
import jax.numpy as jnp
from jax.experimental import pallas as pl
from jax.experimental.pallas import tpu as pltpu


def _layer_kernel(adj_ref, emb_ref, w_ref, out_ref, b_ref, acc_ref,
                  *, n_e_blocks, normalize):
    n = pl.program_id(0)
    e = pl.program_id(1)
    R = adj_ref.shape[0]
    te = adj_ref.shape[2]

    @pl.when(jnp.logical_and(n == 0, e == 0))
    def _compute_b():
        emb = emb_ref[...]
        for r in range(R):
            b_ref[r] = jax.lax.dot_general(
                emb, w_ref[r], (((1,), (1,)), ((), ())),
                preferred_element_type=jnp.float32)

    partial = None
    for r in range(R):
        p = jnp.dot(adj_ref[r], b_ref[r, pl.ds(e * te, te), :],
                    preferred_element_type=jnp.float32)
        partial = p if partial is None else partial + p

    @pl.when(e == 0)
    def _init():
        acc_ref[...] = partial

    @pl.when(e != 0)
    def _accum():
        acc_ref[...] += partial

    @pl.when(e == n_e_blocks - 1)
    def _epilogue():
        a = jnp.maximum(acc_ref[...], 0.0)
        if normalize:
            norm = jnp.sqrt(jnp.sum(a * a, axis=1, keepdims=True))
            a = a / jnp.maximum(norm, 1e-12)
        out_ref[...] = a


def _layer(adj, emb, rel_trans, *, tn, te, normalize):
    R, N, E = adj.shape
    D = emb.shape[1]
    n_n, n_e = N // tn, E // te
    kern = functools.partial(_layer_kernel, n_e_blocks=n_e, normalize=normalize)
    return pl.pallas_call(
        kern,
        grid=(n_n, n_e),
        in_specs=[
            pl.BlockSpec((R, tn, te), lambda n, e: (0, n, e)),
            pl.BlockSpec((N, D), lambda n, e: (0, 0)),
            pl.BlockSpec((R, D, D), lambda n, e: (0, 0, 0)),
        ],
        out_specs=pl.BlockSpec((tn, D), lambda n, e: (n, 0)),
        out_shape=jax.ShapeDtypeStruct((N, D), jnp.float32),
        scratch_shapes=[
            pltpu.VMEM((R, E, D), jnp.float32),
            pltpu.VMEM((tn, D), jnp.float32),
        ],
        compiler_params=pltpu.CompilerParams(
            dimension_semantics=("arbitrary", "arbitrary"),
        ),
    )(adj, emb, rel_trans)


def kernel(adj_mat, ent_emb, rel_trans):
    tn, te = 512, 512
    emb = _layer(adj_mat, ent_emb, rel_trans, tn=tn, te=te, normalize=False)
    emb = _layer(adj_mat, emb, rel_trans, tn=tn, te=te, normalize=True)
    return emb


# speedup vs baseline: 1.0425x; 1.0425x over previous
import functools

import jax
import jax.numpy as jnp
from jax.experimental import pallas as pl
from jax.experimental.pallas import tpu as pltpu


def _layer_kernel(adj_ref, emb_ref, w_ref, out_ref, b_ref, acc_ref,
                  *, n_e_blocks, normalize):
    n = pl.program_id(0)
    e = pl.program_id(1)
    R = adj_ref.shape[0]
    te = adj_ref.shape[2]

    @pl.when(jnp.logical_and(n == 0, e == 0))
    def _compute_b():
        emb = emb_ref[...]
        for r in range(R):
            b_ref[r] = jax.lax.dot_general(
                emb, w_ref[r], (((1,), (1,)), ((), ())),
                preferred_element_type=jnp.float32).astype(jnp.bfloat16)

    partial = None
    for r in range(R):
        p = jnp.dot(adj_ref[r].astype(jnp.bfloat16),
                    b_ref[r, pl.ds(e * te, te), :],
                    preferred_element_type=jnp.float32)
        partial = p if partial is None else partial + p

    @pl.when(e == 0)
    def _init():
        acc_ref[...] = partial

    @pl.when(e != 0)
    def _accum():
        acc_ref[...] += partial

    @pl.when(e == n_e_blocks - 1)
    def _epilogue():
        a = jnp.maximum(acc_ref[...], 0.0)
        if normalize:
            norm = jnp.sqrt(jnp.sum(a * a, axis=1, keepdims=True))
            a = a / jnp.maximum(norm, 1e-12)
        out_ref[...] = a


def _layer(adj, emb, rel_trans, *, tn, te, normalize):
    R, N, E = adj.shape
    D = emb.shape[1]
    n_n, n_e = N // tn, E // te
    kern = functools.partial(_layer_kernel, n_e_blocks=n_e, normalize=normalize)
    return pl.pallas_call(
        kern,
        grid=(n_n, n_e),
        in_specs=[
            pl.BlockSpec((R, tn, te), lambda n, e: (0, n, e)),
            pl.BlockSpec((N, D), lambda n, e: (0, 0)),
            pl.BlockSpec((R, D, D), lambda n, e: (0, 0, 0)),
        ],
        out_specs=pl.BlockSpec((tn, D), lambda n, e: (n, 0)),
        out_shape=jax.ShapeDtypeStruct((N, D), jnp.float32),
        scratch_shapes=[
            pltpu.VMEM((R, E, D), jnp.bfloat16),
            pltpu.VMEM((tn, D), jnp.float32),
        ],
        compiler_params=pltpu.CompilerParams(
            dimension_semantics=("arbitrary", "arbitrary"),
        ),
    )(adj, emb, rel_trans)


def kernel(adj_mat, ent_emb, rel_trans):
    tn, te = 512, 512
    emb = _layer(adj_mat, ent_emb, rel_trans, tn=tn, te=te, normalize=False)
    emb = _layer(adj_mat, emb, rel_trans, tn=tn, te=te, normalize=True)
    return emb


# speedup vs baseline: 1.0770x; 1.0331x over previous
import functools

import jax
import jax.numpy as jnp
from jax.experimental import pallas as pl
from jax.experimental.pallas import tpu as pltpu


def _b_kernel(emb_ref, w_ref, b_ref):
    emb = emb_ref[...]
    for r in range(w_ref.shape[0]):
        b_ref[r] = jax.lax.dot_general(
            emb, w_ref[r], (((1,), (1,)), ((), ())),
            preferred_element_type=jnp.float32).astype(jnp.bfloat16)


def _compute_b(emb, rel_trans):
    R, D, _ = rel_trans.shape
    N = emb.shape[0]
    return pl.pallas_call(
        _b_kernel,
        out_shape=jax.ShapeDtypeStruct((R, N, D), jnp.bfloat16),
    )(emb, rel_trans)


def _layer_kernel(adj_ref, b_ref, out_ref, acc_ref, *, n_r, normalize):
    r = pl.program_id(1)
    p = jnp.dot(adj_ref[0].astype(jnp.bfloat16), b_ref[0],
                preferred_element_type=jnp.float32)

    @pl.when(r == 0)
    def _init():
        acc_ref[...] = p

    @pl.when(r != 0)
    def _accum():
        acc_ref[...] += p

    @pl.when(r == n_r - 1)
    def _epilogue():
        a = jnp.maximum(acc_ref[...], 0.0)
        if normalize:
            norm = jnp.sqrt(jnp.sum(a * a, axis=1, keepdims=True))
            a = a / jnp.maximum(norm, 1e-12)
        out_ref[...] = a


def _layer(adj, b, *, tn, normalize):
    R, N, E = adj.shape
    D = b.shape[2]
    kern = functools.partial(_layer_kernel, n_r=R, normalize=normalize)
    return pl.pallas_call(
        kern,
        grid=(N // tn, R),
        in_specs=[
            pl.BlockSpec((1, tn, E), lambda n, r: (r, n, 0)),
            pl.BlockSpec((1, E, D), lambda n, r: (r, 0, 0)),
        ],
        out_specs=pl.BlockSpec((tn, D), lambda n, r: (n, 0)),
        out_shape=jax.ShapeDtypeStruct((N, D), jnp.float32),
        scratch_shapes=[
            pltpu.VMEM((tn, D), jnp.float32),
        ],
        compiler_params=pltpu.CompilerParams(
            dimension_semantics=("parallel", "arbitrary"),
        ),
    )(adj, b)


def kernel(adj_mat, ent_emb, rel_trans):
    tn = 512
    b1 = _compute_b(ent_emb, rel_trans)
    emb = _layer(adj_mat, b1, tn=tn, normalize=False)
    b2 = _compute_b(emb, rel_trans)
    emb = _layer(adj_mat, b2, tn=tn, normalize=True)
    return emb


# speedup vs baseline: 1.0963x; 1.0179x over previous
import functools

import jax
import jax.numpy as jnp
from jax.experimental import pallas as pl
from jax.experimental.pallas import tpu as pltpu


def _b_kernel(emb_ref, w_ref, b_ref):
    emb = emb_ref[...]
    for r in range(w_ref.shape[0]):
        b_ref[r] = jax.lax.dot_general(
            emb, w_ref[r], (((1,), (1,)), ((), ())),
            preferred_element_type=jnp.float32).astype(jnp.bfloat16)


def _compute_b(emb, rel_trans):
    R, D, _ = rel_trans.shape
    N = emb.shape[0]
    return pl.pallas_call(
        _b_kernel,
        out_shape=jax.ShapeDtypeStruct((R, N, D), jnp.bfloat16),
    )(emb, rel_trans)


def _layer_kernel(adj_ref, b_ref, out_ref, acc_ref, *, n_r, normalize):
    r = pl.program_id(1)
    p = jnp.dot(adj_ref[0].astype(jnp.bfloat16), b_ref[0],
                preferred_element_type=jnp.float32)

    @pl.when(r == 0)
    def _init():
        acc_ref[...] = p

    @pl.when(r != 0)
    def _accum():
        acc_ref[...] += p

    @pl.when(r == n_r - 1)
    def _epilogue():
        a = jnp.maximum(acc_ref[...], 0.0)
        if normalize:
            norm = jnp.sqrt(jnp.sum(a * a, axis=1, keepdims=True))
            a = a / jnp.maximum(norm, 1e-12)
        out_ref[...] = a


def _layer(adj, b, *, tn, normalize):
    R, N, E = adj.shape
    D = b.shape[2]
    kern = functools.partial(_layer_kernel, n_r=R, normalize=normalize)
    return pl.pallas_call(
        kern,
        grid=(N // tn, R),
        in_specs=[
            pl.BlockSpec((1, tn, E), lambda n, r: (r, n, 0)),
            pl.BlockSpec((1, E, D), lambda n, r: (r, 0, 0)),
        ],
        out_specs=pl.BlockSpec((tn, D), lambda n, r: (n, 0)),
        out_shape=jax.ShapeDtypeStruct((N, D), jnp.float32),
        scratch_shapes=[
            pltpu.VMEM((tn, D), jnp.float32),
        ],
        compiler_params=pltpu.CompilerParams(
            dimension_semantics=("parallel", "arbitrary"),
        ),
    )(adj, b)


def kernel(adj_mat, ent_emb, rel_trans):
    tn = 1024
    b1 = _compute_b(ent_emb, rel_trans)
    emb = _layer(adj_mat, b1, tn=tn, normalize=False)
    b2 = _compute_b(emb, rel_trans)
    emb = _layer(adj_mat, b2, tn=tn, normalize=True)
    return emb
